# Initial kernel scaffold; baseline (speedup 1.0000x reference)
import jax
import jax.numpy as jnp
from jax.experimental import pallas as pl


def kernel(x, edge_index, Wl1, bl1, Wr1, g1, b1, Wl2, bl2, Wr2, g2, b2):
    raise NotImplementedError("write your pallas kernel here")


# speedup vs baseline: 4.3077x; 4.3077x over previous
import functools

import jax
import jax.numpy as jnp
from jax import lax
from jax.experimental import pallas as pl
from jax.experimental.pallas import tpu as pltpu
from jax.experimental.pallas import tpu_sc as plsc

N = 10000
E = 320000
D = 128

NC = 2
NS = 16
NW = NC * NS
EPW = E // NW
C = 80
CPW = EPW // C
RPT = 624
TAIL = N - NS * RPT

_MESH = plsc.VectorSubcoreMesh(core_axis_name="c", subcore_axis_name="s",
                               num_cores=NC, num_subcores=NS)


def _init_shared(zeros_nw, acc_sh, s):
  r0 = s * RPT
  pltpu.sync_copy(zeros_nw.at[pl.ds(r0, RPT)], acc_sh.at[pl.ds(r0, RPT)])

  @pl.when(s == 0)
  def _():
    pltpu.sync_copy(zeros_nw.at[pl.ds(NS * RPT, TAIL)],
                    acc_sh.at[pl.ds(NS * RPT, TAIL)])


def _writeback(acc_sh, out_acc, c, s):
  r0 = s * RPT
  o0 = c * N + r0
  pltpu.sync_copy(acc_sh.at[pl.ds(r0, RPT)], out_acc.at[pl.ds(o0, RPT)])

  @pl.when(s == 0)
  def _():
    pltpu.sync_copy(acc_sh.at[pl.ds(NS * RPT, TAIL)],
                    out_acc.at[pl.ds(c * N + NS * RPT, TAIL)])


def _agg_body(y, src, dst, zeros_nw, out_acc, src_idx, dst_idx, rows, sem,
              acc_sh):
  c = lax.axis_index("c")
  s = lax.axis_index("s")
  w = s * NC + c
  _init_shared(zeros_nw, acc_sh, s)
  plsc.subcore_barrier()

  e0 = w * EPW

  def step(i, carry):
    base = e0 + i * C
    pltpu.sync_copy(src.at[pl.ds(base, C)], src_idx)
    pltpu.sync_copy(dst.at[pl.ds(base, C)], dst_idx)
    pltpu.async_copy(y.at[src_idx], rows, sem).wait()
    pltpu.sync_copy(rows, acc_sh.at[dst_idx], add=True)
    return carry

  lax.fori_loop(0, CPW, step, 0)
  plsc.subcore_barrier()
  _writeback(acc_sh, out_acc, c, s)


_agg = pl.kernel(
    _agg_body,
    out_type=jax.ShapeDtypeStruct((NC * N, D), jnp.float32),
    mesh=_MESH,
    scratch_types=(
        pltpu.VMEM((C,), jnp.int32),
        pltpu.VMEM((C,), jnp.int32),
        pltpu.VMEM((C, D), jnp.float32),
        pltpu.SemaphoreType.DMA,
        pltpu.VMEM_SHARED((N, D), jnp.float32),
    ))


def _cnt_body(dst, zeros_nw, ones_cd, out_cnt, dst_idx, ones_v, cnt_sh):
  c = lax.axis_index("c")
  s = lax.axis_index("s")
  w = s * NC + c
  _init_shared(zeros_nw, cnt_sh, s)
  pltpu.sync_copy(ones_cd, ones_v)
  plsc.subcore_barrier()

  e0 = w * EPW

  def step(i, carry):
    base = e0 + i * C
    pltpu.sync_copy(dst.at[pl.ds(base, C)], dst_idx)
    pltpu.sync_copy(ones_v, cnt_sh.at[dst_idx], add=True)
    return carry

  lax.fori_loop(0, CPW, step, 0)
  plsc.subcore_barrier()
  _writeback(cnt_sh, out_cnt, c, s)


_cnt_agg = pl.kernel(
    _cnt_body,
    out_type=jax.ShapeDtypeStruct((NC * N, D), jnp.float32),
    mesh=_MESH,
    scratch_types=(
        pltpu.VMEM((C,), jnp.int32),
        pltpu.VMEM((C, D), jnp.float32),
        pltpu.VMEM_SHARED((N, D), jnp.float32),
    ))


def _dense1_body(a0_ref, a1_ref, c0_ref, c1_ref, x_ref, wl_ref, wr_ref,
                 bl_ref, g_ref, b_ref, o_ref, cnt_ref):
  acc = a0_ref[...] + a1_ref[...]
  cnt = jnp.maximum(c0_ref[:, 0:1] + c1_ref[:, 0:1], 1.0)
  mean = acc / cnt
  xb = x_ref[...]
  h = (jax.lax.dot(mean, wl_ref[...], precision=lax.Precision.HIGHEST,
                   preferred_element_type=jnp.float32)
       + bl_ref[...]
       + jax.lax.dot(xb, wr_ref[...], precision=lax.Precision.HIGHEST,
                     preferred_element_type=jnp.float32))
  mu = jnp.mean(h, axis=-1, keepdims=True)
  d = h - mu
  var = jnp.mean(d * d, axis=-1, keepdims=True)
  hn = d * jax.lax.rsqrt(var + 1e-5) * g_ref[...] + b_ref[...]
  o_ref[...] = jnp.maximum(hn, 0.0) + xb
  cnt_ref[...] = jnp.broadcast_to(cnt, cnt_ref.shape)


def _dense2_body(a0_ref, a1_ref, cnt_ref, x_ref, wl_ref, wr_ref, bl_ref,
                 g_ref, b_ref, o_ref):
  acc = a0_ref[...] + a1_ref[...]
  mean = acc / cnt_ref[:, 0:1]
  xb = x_ref[...]
  h = (jax.lax.dot(mean, wl_ref[...], precision=lax.Precision.HIGHEST,
                   preferred_element_type=jnp.float32)
       + bl_ref[...]
       + jax.lax.dot(xb, wr_ref[...], precision=lax.Precision.HIGHEST,
                     preferred_element_type=jnp.float32))
  mu = jnp.mean(h, axis=-1, keepdims=True)
  d = h - mu
  var = jnp.mean(d * d, axis=-1, keepdims=True)
  hn = d * jax.lax.rsqrt(var + 1e-5) * g_ref[...] + b_ref[...]
  o_ref[...] = hn + xb


R = 1000
NB = N // R

_W_SPEC = pl.BlockSpec((D, D), lambda i: (0, 0))
_B_SPEC = pl.BlockSpec((1, D), lambda i: (0, 0))
_ROW_SPEC = pl.BlockSpec((R, D), lambda i: (i, 0))
_LO_SPEC = pl.BlockSpec((R, D), lambda i: (i, 0))
_HI_SPEC = pl.BlockSpec((R, D), lambda i: (NB + i, 0))

_dense1 = pl.pallas_call(
    _dense1_body,
    grid=(NB,),
    in_specs=[
        _LO_SPEC, _HI_SPEC, _LO_SPEC, _HI_SPEC, _ROW_SPEC,
        _W_SPEC, _W_SPEC, _B_SPEC, _B_SPEC, _B_SPEC,
    ],
    out_specs=[
        _ROW_SPEC,
        pl.BlockSpec((R, 8), lambda i: (i, 0)),
    ],
    out_shape=[
        jax.ShapeDtypeStruct((N, D), jnp.float32),
        jax.ShapeDtypeStruct((N, 8), jnp.float32),
    ],
)

_dense2 = pl.pallas_call(
    _dense2_body,
    grid=(NB,),
    in_specs=[
        _LO_SPEC, _HI_SPEC,
        pl.BlockSpec((R, 8), lambda i: (i, 0)),
        _ROW_SPEC,
        _W_SPEC, _W_SPEC, _B_SPEC, _B_SPEC, _B_SPEC,
    ],
    out_specs=_ROW_SPEC,
    out_shape=jax.ShapeDtypeStruct((N, D), jnp.float32),
)


@jax.jit
def kernel(x, edge_index, Wl1, bl1, Wr1, g1, b1, Wl2, bl2, Wr2, g2, b2):
  src = edge_index[0]
  dst = edge_index[1]
  zeros_nd = jnp.zeros((N, D), jnp.float32)
  ones_cd = jnp.ones((C, D), jnp.float32)

  cntp = _cnt_agg(dst, zeros_nd, ones_cd)
  acc1 = _agg(x, src, dst, zeros_nd)
  h, cnt8 = _dense1(acc1, acc1, cntp, cntp, x, Wl1.T, Wr1.T,
                    bl1.reshape(1, D), g1.reshape(1, D), b1.reshape(1, D))
  acc2 = _agg(h, src, dst, zeros_nd)
  out = _dense2(acc2, acc2, cnt8, h, Wl2.T, Wr2.T, bl2.reshape(1, D),
                g2.reshape(1, D), b2.reshape(1, D))
  return out


# speedup vs baseline: 6.2060x; 1.4407x over previous
import functools

import jax
import jax.numpy as jnp
from jax import lax
from jax.experimental import pallas as pl
from jax.experimental.pallas import tpu as pltpu
from jax.experimental.pallas import tpu_sc as plsc

N = 10000
E = 320000
D = 128

NC = 2
NS = 16
NW = NC * NS
EPW = E // NW
C = 80
CPW = EPW // C
RPT = 624
TAIL = N - NS * RPT

_MESH = plsc.VectorSubcoreMesh(core_axis_name="c", subcore_axis_name="s",
                               num_cores=NC, num_subcores=NS)


def _init_shared(zeros_nw, acc_sh, s):
  r0 = s * RPT
  pltpu.sync_copy(zeros_nw.at[pl.ds(r0, RPT)], acc_sh.at[pl.ds(r0, RPT)])

  @pl.when(s == 0)
  def _():
    pltpu.sync_copy(zeros_nw.at[pl.ds(NS * RPT, TAIL)],
                    acc_sh.at[pl.ds(NS * RPT, TAIL)])


def _writeback(acc_sh, out_acc, c, s):
  r0 = s * RPT
  o0 = c * N + r0
  pltpu.sync_copy(acc_sh.at[pl.ds(r0, RPT)], out_acc.at[pl.ds(o0, RPT)])

  @pl.when(s == 0)
  def _():
    pltpu.sync_copy(acc_sh.at[pl.ds(NS * RPT, TAIL)],
                    out_acc.at[pl.ds(c * N + NS * RPT, TAIL)])


def _agg_body(y, src, dst, zeros_nw, out_acc, idx, buf0, buf1, sem0, sem1,
              acc_sh):
  c = lax.axis_index("c")
  s = lax.axis_index("s")
  w = s * NC + c
  _init_shared(zeros_nw, acc_sh, s)
  plsc.subcore_barrier()

  e0 = w * EPW

  def stage(ci, b):
    base = e0 + ci * C
    pltpu.sync_copy(src.at[pl.ds(base, C)], idx.at[b])
    pltpu.sync_copy(dst.at[pl.ds(base, C)], idx.at[2 + b])

  def gather(b, buf, sem):
    pltpu.async_copy(y.at[idx.at[b]], buf, sem)

  def finish(b, buf, sem):
    pltpu.make_async_copy(y.at[idx.at[b]], buf, sem).wait()
    pltpu.sync_copy(buf, acc_sh.at[idx.at[2 + b]], add=True)

  stage(0, 0)
  gather(0, buf0, sem0)

  def step(i, carry):
    c0 = 2 * i
    stage(c0 + 1, 1)
    gather(1, buf1, sem1)
    finish(0, buf0, sem0)
    stage(c0 + 2, 0)
    gather(0, buf0, sem0)
    finish(1, buf1, sem1)
    return carry

  lax.fori_loop(0, (CPW - 1) // 2, step, 0)
  finish(0, buf0, sem0)
  plsc.subcore_barrier()
  _writeback(acc_sh, out_acc, c, s)


_agg = pl.kernel(
    _agg_body,
    out_type=jax.ShapeDtypeStruct((NC * N, D), jnp.float32),
    mesh=_MESH,
    scratch_types=(
        pltpu.VMEM((4, C), jnp.int32),
        pltpu.VMEM((C, D), jnp.float32),
        pltpu.VMEM((C, D), jnp.float32),
        pltpu.SemaphoreType.DMA,
        pltpu.SemaphoreType.DMA,
        pltpu.VMEM_SHARED((N, D), jnp.float32),
    ))


def _cnt_body(dst, zeros_nw, ones_cd, out_cnt, dst_idx, ones_v, cnt_sh):
  c = lax.axis_index("c")
  s = lax.axis_index("s")
  w = s * NC + c
  _init_shared(zeros_nw, cnt_sh, s)
  pltpu.sync_copy(ones_cd, ones_v)
  plsc.subcore_barrier()

  e0 = w * EPW

  def step(i, carry):
    base = e0 + i * C
    pltpu.sync_copy(dst.at[pl.ds(base, C)], dst_idx)
    pltpu.sync_copy(ones_v, cnt_sh.at[dst_idx], add=True)
    return carry

  lax.fori_loop(0, CPW, step, 0)
  plsc.subcore_barrier()
  _writeback(cnt_sh, out_cnt, c, s)


_cnt_agg = pl.kernel(
    _cnt_body,
    out_type=jax.ShapeDtypeStruct((NC * N, D), jnp.float32),
    mesh=_MESH,
    scratch_types=(
        pltpu.VMEM((C,), jnp.int32),
        pltpu.VMEM((C, D), jnp.float32),
        pltpu.VMEM_SHARED((N, D), jnp.float32),
    ))


def _dense1_body(a0_ref, a1_ref, c0_ref, c1_ref, x_ref, wl_ref, wr_ref,
                 bl_ref, g_ref, b_ref, o_ref, cnt_ref):
  acc = a0_ref[...] + a1_ref[...]
  cnt = jnp.maximum(c0_ref[:, 0:1] + c1_ref[:, 0:1], 1.0)
  mean = acc / cnt
  xb = x_ref[...]
  h = (jax.lax.dot(mean, wl_ref[...], precision=lax.Precision.HIGHEST,
                   preferred_element_type=jnp.float32)
       + bl_ref[...]
       + jax.lax.dot(xb, wr_ref[...], precision=lax.Precision.HIGHEST,
                     preferred_element_type=jnp.float32))
  mu = jnp.mean(h, axis=-1, keepdims=True)
  d = h - mu
  var = jnp.mean(d * d, axis=-1, keepdims=True)
  hn = d * jax.lax.rsqrt(var + 1e-5) * g_ref[...] + b_ref[...]
  o_ref[...] = jnp.maximum(hn, 0.0) + xb
  cnt_ref[...] = jnp.broadcast_to(cnt, cnt_ref.shape)


def _dense2_body(a0_ref, a1_ref, cnt_ref, x_ref, wl_ref, wr_ref, bl_ref,
                 g_ref, b_ref, o_ref):
  acc = a0_ref[...] + a1_ref[...]
  mean = acc / cnt_ref[:, 0:1]
  xb = x_ref[...]
  h = (jax.lax.dot(mean, wl_ref[...], precision=lax.Precision.HIGHEST,
                   preferred_element_type=jnp.float32)
       + bl_ref[...]
       + jax.lax.dot(xb, wr_ref[...], precision=lax.Precision.HIGHEST,
                     preferred_element_type=jnp.float32))
  mu = jnp.mean(h, axis=-1, keepdims=True)
  d = h - mu
  var = jnp.mean(d * d, axis=-1, keepdims=True)
  hn = d * jax.lax.rsqrt(var + 1e-5) * g_ref[...] + b_ref[...]
  o_ref[...] = hn + xb


R = 1000
NB = N // R

_W_SPEC = pl.BlockSpec((D, D), lambda i: (0, 0))
_B_SPEC = pl.BlockSpec((1, D), lambda i: (0, 0))
_ROW_SPEC = pl.BlockSpec((R, D), lambda i: (i, 0))
_LO_SPEC = pl.BlockSpec((R, D), lambda i: (i, 0))
_HI_SPEC = pl.BlockSpec((R, D), lambda i: (NB + i, 0))

_dense1 = pl.pallas_call(
    _dense1_body,
    grid=(NB,),
    in_specs=[
        _LO_SPEC, _HI_SPEC, _LO_SPEC, _HI_SPEC, _ROW_SPEC,
        _W_SPEC, _W_SPEC, _B_SPEC, _B_SPEC, _B_SPEC,
    ],
    out_specs=[
        _ROW_SPEC,
        pl.BlockSpec((R, 8), lambda i: (i, 0)),
    ],
    out_shape=[
        jax.ShapeDtypeStruct((N, D), jnp.float32),
        jax.ShapeDtypeStruct((N, 8), jnp.float32),
    ],
)

_dense2 = pl.pallas_call(
    _dense2_body,
    grid=(NB,),
    in_specs=[
        _LO_SPEC, _HI_SPEC,
        pl.BlockSpec((R, 8), lambda i: (i, 0)),
        _ROW_SPEC,
        _W_SPEC, _W_SPEC, _B_SPEC, _B_SPEC, _B_SPEC,
    ],
    out_specs=_ROW_SPEC,
    out_shape=jax.ShapeDtypeStruct((N, D), jnp.float32),
)


@jax.jit
def kernel(x, edge_index, Wl1, bl1, Wr1, g1, b1, Wl2, bl2, Wr2, g2, b2):
  src = edge_index[0]
  dst = edge_index[1]
  zeros_nd = jnp.zeros((N, D), jnp.float32)
  ones_cd = jnp.ones((C, D), jnp.float32)

  cntp = _cnt_agg(dst, zeros_nd, ones_cd)
  acc1 = _agg(x, src, dst, zeros_nd)
  h, cnt8 = _dense1(acc1, acc1, cntp, cntp, x, Wl1.T, Wr1.T,
                    bl1.reshape(1, D), g1.reshape(1, D), b1.reshape(1, D))
  acc2 = _agg(h, src, dst, zeros_nd)
  out = _dense2(acc2, acc2, cnt8, h, Wl2.T, Wr2.T, bl2.reshape(1, D),
                g2.reshape(1, D), b2.reshape(1, D))
  return out


# speedup vs baseline: 7.6428x; 1.2315x over previous
import functools

import jax
import jax.numpy as jnp
from jax import lax
from jax.experimental import pallas as pl
from jax.experimental.pallas import tpu as pltpu
from jax.experimental.pallas import tpu_sc as plsc

N = 10000
E = 320000
D = 128

NC = 2
NS = 16
NW = NC * NS
EPW = E // NW
C = 80
CPW = EPW // C
RPT = 624
TAIL = N - NS * RPT

_MESH = plsc.VectorSubcoreMesh(core_axis_name="c", subcore_axis_name="s",
                               num_cores=NC, num_subcores=NS)


def _init_shared(zeros_nw, acc_sh, s):
  r0 = s * RPT
  pltpu.sync_copy(zeros_nw.at[pl.ds(r0, RPT)], acc_sh.at[pl.ds(r0, RPT)])

  @pl.when(s == 0)
  def _():
    pltpu.sync_copy(zeros_nw.at[pl.ds(NS * RPT, TAIL)],
                    acc_sh.at[pl.ds(NS * RPT, TAIL)])


def _writeback(acc_sh, out_acc, c, s):
  r0 = s * RPT
  o0 = c * N + r0
  pltpu.sync_copy(acc_sh.at[pl.ds(r0, RPT)], out_acc.at[pl.ds(o0, RPT)])

  @pl.when(s == 0)
  def _():
    pltpu.sync_copy(acc_sh.at[pl.ds(NS * RPT, TAIL)],
                    out_acc.at[pl.ds(c * N + NS * RPT, TAIL)])


def _agg_body(y, src2, dst3, zeros_nw, out_acc, idx_s, idx_d, buf0, buf1,
              g0, g1, s0, s1, acc_sh):
  c = lax.axis_index("c")
  s = lax.axis_index("s")
  w = s * NC + c
  _init_shared(zeros_nw, acc_sh, s)
  pltpu.sync_copy(src2.at[w], idx_s)
  pltpu.sync_copy(dst3.at[w], idx_d)
  plsc.subcore_barrier()

  def gather(k, buf, sem):
    pltpu.async_copy(y.at[idx_s.at[pl.ds(k * C, C)]], buf, sem)

  def scatter(k, buf, sem):
    pltpu.async_copy(buf, acc_sh.at[idx_d.at[k]], sem, add=True)

  def wait_g(buf, sem):
    pltpu.make_async_copy(y.at[idx_s.at[pl.ds(0, C)]], buf, sem).wait()

  def wait_s(buf, sem):
    pltpu.make_async_copy(buf, acc_sh.at[idx_d.at[0]], sem).wait()

  gather(0, buf0, g0)
  gather(1, buf1, g1)

  def step(i, carry):
    c0 = 2 * i
    wait_g(buf0, g0)
    scatter(c0, buf0, s0)
    wait_g(buf1, g1)
    scatter(c0 + 1, buf1, s1)
    wait_s(buf0, s0)
    gather(c0 + 2, buf0, g0)
    wait_s(buf1, s1)
    gather(c0 + 3, buf1, g1)
    return carry

  lax.fori_loop(0, (CPW - 3) // 2, step, 0)
  wait_g(buf0, g0)
  scatter(CPW - 3, buf0, s0)
  wait_g(buf1, g1)
  scatter(CPW - 2, buf1, s1)
  wait_s(buf0, s0)
  gather(CPW - 1, buf0, g0)
  wait_g(buf0, g0)
  scatter(CPW - 1, buf0, s0)
  wait_s(buf0, s0)
  wait_s(buf1, s1)
  plsc.subcore_barrier()
  _writeback(acc_sh, out_acc, c, s)


_agg = pl.kernel(
    _agg_body,
    out_type=jax.ShapeDtypeStruct((NC * N, D), jnp.float32),
    mesh=_MESH,
    scratch_types=(
        pltpu.VMEM((EPW,), jnp.int32),
        pltpu.VMEM((CPW, C), jnp.int32),
        pltpu.VMEM((C, D), jnp.float32),
        pltpu.VMEM((C, D), jnp.float32),
        pltpu.SemaphoreType.DMA,
        pltpu.SemaphoreType.DMA,
        pltpu.SemaphoreType.DMA,
        pltpu.SemaphoreType.DMA,
        pltpu.VMEM_SHARED((N, D), jnp.float32),
    ))


def _cnt_body(dst3, zeros_nw, ones_cd, out_cnt, idx_d, ones_v, s0, s1,
              cnt_sh):
  c = lax.axis_index("c")
  s = lax.axis_index("s")
  w = s * NC + c
  _init_shared(zeros_nw, cnt_sh, s)
  pltpu.sync_copy(ones_cd, ones_v)
  pltpu.sync_copy(dst3.at[w], idx_d)
  plsc.subcore_barrier()

  def scatter(k, sem):
    pltpu.async_copy(ones_v, cnt_sh.at[idx_d.at[k]], sem, add=True)

  def wait_s(sem):
    pltpu.make_async_copy(ones_v, cnt_sh.at[idx_d.at[0]], sem).wait()

  scatter(0, s0)
  scatter(1, s1)

  def step(i, carry):
    c0 = 2 * i
    wait_s(s0)
    scatter(c0 + 2, s0)
    wait_s(s1)
    scatter(c0 + 3, s1)
    return carry

  lax.fori_loop(0, (CPW - 3) // 2, step, 0)
  wait_s(s0)
  scatter(CPW - 1, s0)
  wait_s(s0)
  wait_s(s1)
  plsc.subcore_barrier()
  _writeback(cnt_sh, out_cnt, c, s)


_cnt_agg = pl.kernel(
    _cnt_body,
    out_type=jax.ShapeDtypeStruct((NC * N, D), jnp.float32),
    mesh=_MESH,
    scratch_types=(
        pltpu.VMEM((CPW, C), jnp.int32),
        pltpu.VMEM((C, D), jnp.float32),
        pltpu.SemaphoreType.DMA,
        pltpu.SemaphoreType.DMA,
        pltpu.VMEM_SHARED((N, D), jnp.float32),
    ))


def _dense1_body(a0_ref, a1_ref, c0_ref, c1_ref, x_ref, wl_ref, wr_ref,
                 bl_ref, g_ref, b_ref, o_ref, cnt_ref):
  acc = a0_ref[...] + a1_ref[...]
  cnt = jnp.maximum(c0_ref[:, 0:1] + c1_ref[:, 0:1], 1.0)
  mean = acc / cnt
  xb = x_ref[...]
  h = (jax.lax.dot(mean, wl_ref[...], precision=lax.Precision.HIGHEST,
                   preferred_element_type=jnp.float32)
       + bl_ref[...]
       + jax.lax.dot(xb, wr_ref[...], precision=lax.Precision.HIGHEST,
                     preferred_element_type=jnp.float32))
  mu = jnp.mean(h, axis=-1, keepdims=True)
  d = h - mu
  var = jnp.mean(d * d, axis=-1, keepdims=True)
  hn = d * jax.lax.rsqrt(var + 1e-5) * g_ref[...] + b_ref[...]
  o_ref[...] = jnp.maximum(hn, 0.0) + xb
  cnt_ref[...] = jnp.broadcast_to(cnt, cnt_ref.shape)


def _dense2_body(a0_ref, a1_ref, cnt_ref, x_ref, wl_ref, wr_ref, bl_ref,
                 g_ref, b_ref, o_ref):
  acc = a0_ref[...] + a1_ref[...]
  mean = acc / cnt_ref[:, 0:1]
  xb = x_ref[...]
  h = (jax.lax.dot(mean, wl_ref[...], precision=lax.Precision.HIGHEST,
                   preferred_element_type=jnp.float32)
       + bl_ref[...]
       + jax.lax.dot(xb, wr_ref[...], precision=lax.Precision.HIGHEST,
                     preferred_element_type=jnp.float32))
  mu = jnp.mean(h, axis=-1, keepdims=True)
  d = h - mu
  var = jnp.mean(d * d, axis=-1, keepdims=True)
  hn = d * jax.lax.rsqrt(var + 1e-5) * g_ref[...] + b_ref[...]
  o_ref[...] = hn + xb


R = 1000
NB = N // R

_W_SPEC = pl.BlockSpec((D, D), lambda i: (0, 0))
_B_SPEC = pl.BlockSpec((1, D), lambda i: (0, 0))
_ROW_SPEC = pl.BlockSpec((R, D), lambda i: (i, 0))
_LO_SPEC = pl.BlockSpec((R, D), lambda i: (i, 0))
_HI_SPEC = pl.BlockSpec((R, D), lambda i: (NB + i, 0))

_dense1 = pl.pallas_call(
    _dense1_body,
    grid=(NB,),
    in_specs=[
        _LO_SPEC, _HI_SPEC, _LO_SPEC, _HI_SPEC, _ROW_SPEC,
        _W_SPEC, _W_SPEC, _B_SPEC, _B_SPEC, _B_SPEC,
    ],
    out_specs=[
        _ROW_SPEC,
        pl.BlockSpec((R, 8), lambda i: (i, 0)),
    ],
    out_shape=[
        jax.ShapeDtypeStruct((N, D), jnp.float32),
        jax.ShapeDtypeStruct((N, 8), jnp.float32),
    ],
)

_dense2 = pl.pallas_call(
    _dense2_body,
    grid=(NB,),
    in_specs=[
        _LO_SPEC, _HI_SPEC,
        pl.BlockSpec((R, 8), lambda i: (i, 0)),
        _ROW_SPEC,
        _W_SPEC, _W_SPEC, _B_SPEC, _B_SPEC, _B_SPEC,
    ],
    out_specs=_ROW_SPEC,
    out_shape=jax.ShapeDtypeStruct((N, D), jnp.float32),
)


@jax.jit
def kernel(x, edge_index, Wl1, bl1, Wr1, g1, b1, Wl2, bl2, Wr2, g2, b2):
  src = edge_index[0]
  dst = edge_index[1]
  src2 = src.reshape(NW, EPW)
  dst3 = dst.reshape(NW, CPW, C)
  zeros_nd = jnp.zeros((N, D), jnp.float32)
  ones_cd = jnp.ones((C, D), jnp.float32)

  cntp = _cnt_agg(dst3, zeros_nd, ones_cd)
  acc1 = _agg(x, src2, dst3, zeros_nd)
  h, cnt8 = _dense1(acc1, acc1, cntp, cntp, x, Wl1.T, Wr1.T,
                    bl1.reshape(1, D), g1.reshape(1, D), b1.reshape(1, D))
  acc2 = _agg(h, src2, dst3, zeros_nd)
  out = _dense2(acc2, acc2, cnt8, h, Wl2.T, Wr2.T, bl2.reshape(1, D),
                g2.reshape(1, D), b2.reshape(1, D))
  return out
